# Initial kernel scaffold; baseline (speedup 1.0000x reference)
import jax
import jax.numpy as jnp
from jax.experimental import pallas as pl


def kernel(x, edge_index, batch, W0, b0, W1, b1, W2, b2, Wl, bl):
    raise NotImplementedError("write your pallas kernel here")


# speedup vs baseline: 13.0279x; 13.0279x over previous
import functools

import jax
import jax.numpy as jnp
from jax import lax
from jax.experimental import pallas as pl
from jax.experimental.pallas import tpu as pltpu
from jax.experimental.pallas import tpu_sc as plsc

N = 10000
E = 320000
D = 128
H = 128
G = 64

NC = 2
NS = 16
NW = NC * NS
EPW = E // NW
CHUNK = 128
NFULL = EPW // CHUNK
TAIL = EPW - NFULL * CHUNK
NP = 10240
RPT = NP // NS
DEGW = 128

BN = 1000
GRID = N // BN


@functools.cache
def _sc_kernels():
    mesh = plsc.VectorSubcoreMesh(
        core_axis_name="c", subcore_axis_name="s", num_cores=NC, num_subcores=NS
    )

    @functools.partial(
        pl.kernel,
        out_type=jax.ShapeDtypeStruct((NC, NP, DEGW), jnp.float32),
        mesh=mesh,
        scratch_types=[
            pltpu.VMEM((CHUNK,), jnp.int32),
            pltpu.VMEM((TAIL,), jnp.int32),
            pltpu.VMEM((CHUNK, DEGW), jnp.float32),
            pltpu.VMEM_SHARED((NP, DEGW), jnp.float32),
        ],
    )
    def deg_fn(dst_h, ones_h, zeros_h, out, idx_v, idxt_v, ones_v, acc):
        c = lax.axis_index("c")
        s = lax.axis_index("s")
        wid = c * NS + s
        pltpu.sync_copy(ones_h, ones_v)
        pltpu.sync_copy(zeros_h, acc.at[pl.ds(s * RPT, RPT)])
        plsc.subcore_barrier()
        base = wid * EPW

        def body(i, carry):
            off = pl.multiple_of(base + i * CHUNK, 8)
            pltpu.sync_copy(dst_h.at[pl.ds(off, CHUNK)], idx_v)
            pltpu.sync_copy(ones_v, acc.at[idx_v], add=True)
            return carry

        lax.fori_loop(0, NFULL, body, 0)
        offt = pl.multiple_of(base + NFULL * CHUNK, 8)
        pltpu.sync_copy(dst_h.at[pl.ds(offt, TAIL)], idxt_v)
        pltpu.sync_copy(ones_v.at[pl.ds(0, TAIL)], acc.at[idxt_v], add=True)
        plsc.subcore_barrier()
        pltpu.sync_copy(acc.at[pl.ds(s * RPT, RPT)], out.at[c, pl.ds(s * RPT, RPT)])

    @functools.partial(
        pl.kernel,
        out_type=jax.ShapeDtypeStruct((NC, NP, H), jnp.float32),
        mesh=mesh,
        scratch_types=[
            pltpu.VMEM((CHUNK,), jnp.int32),
            pltpu.VMEM((CHUNK,), jnp.int32),
            pltpu.VMEM((TAIL,), jnp.int32),
            pltpu.VMEM((TAIL,), jnp.int32),
            pltpu.VMEM((CHUNK, H), jnp.float32),
            pltpu.VMEM((TAIL, H), jnp.float32),
            pltpu.VMEM_SHARED((NP, H), jnp.float32),
            pltpu.SemaphoreType.DMA,
        ],
    )
    def agg_fn(g, src_h, dst_h, zeros_h, out, idx_s, idx_d, idxt_s, idxt_d, rows, rowst,
               acc, sem):
        c = lax.axis_index("c")
        s = lax.axis_index("s")
        wid = c * NS + s
        pltpu.sync_copy(zeros_h, acc.at[pl.ds(s * RPT, RPT)])
        plsc.subcore_barrier()
        base = wid * EPW

        def body(i, carry):
            off = pl.multiple_of(base + i * CHUNK, 8)
            pltpu.sync_copy(src_h.at[pl.ds(off, CHUNK)], idx_s)
            pltpu.sync_copy(dst_h.at[pl.ds(off, CHUNK)], idx_d)
            pltpu.async_copy(g.at[idx_s], rows, sem).wait()
            pltpu.sync_copy(rows, acc.at[idx_d], add=True)
            return carry

        lax.fori_loop(0, NFULL, body, 0)
        offt = pl.multiple_of(base + NFULL * CHUNK, 8)
        pltpu.sync_copy(src_h.at[pl.ds(offt, TAIL)], idxt_s)
        pltpu.sync_copy(dst_h.at[pl.ds(offt, TAIL)], idxt_d)
        pltpu.async_copy(g.at[idxt_s], rowst, sem).wait()
        pltpu.sync_copy(rowst, acc.at[idxt_d], add=True)
        plsc.subcore_barrier()
        pltpu.sync_copy(acc.at[pl.ds(s * RPT, RPT)], out.at[c, pl.ds(s * RPT, RPT)])

    return deg_fn, agg_fn


def _pre_body(deg_ref, x_ref, w_ref, dinv_ref, g_ref):
    deg = 1.0 + deg_ref[0, :, 0:1] + deg_ref[1, :, 0:1]
    dinv = lax.rsqrt(deg)
    dinv_ref[...] = dinv
    g_ref[...] = dinv * jnp.dot(
        x_ref[...], w_ref[...], preferred_element_type=jnp.float32,
        precision=lax.Precision.HIGHEST
    )


def _pre(deg, x, w):
    return pl.pallas_call(
        _pre_body,
        grid=(GRID,),
        in_specs=[
            pl.BlockSpec((NC, BN, DEGW), lambda i: (0, i, 0)),
            pl.BlockSpec((BN, D), lambda i: (i, 0)),
            pl.BlockSpec((D, H), lambda i: (0, 0)),
        ],
        out_specs=[
            pl.BlockSpec((BN, 1), lambda i: (i, 0)),
            pl.BlockSpec((BN, H), lambda i: (i, 0)),
        ],
        out_shape=[
            jax.ShapeDtypeStruct((N, 1), jnp.float32),
            jax.ShapeDtypeStruct((N, H), jnp.float32),
        ],
    )(deg, x, w)


def _layer_body(acc_ref, g_ref, dinv_ref, b_ref, w_ref, out_ref):
    t = acc_ref[0] + acc_ref[1] + g_ref[...]
    h = jnp.maximum(dinv_ref[...] * t + b_ref[...], 0.0)
    out_ref[...] = dinv_ref[...] * jnp.dot(
        h, w_ref[...], preferred_element_type=jnp.float32,
        precision=lax.Precision.HIGHEST
    )


def _layer(acc, g, dinv, b, w):
    return pl.pallas_call(
        _layer_body,
        grid=(GRID,),
        in_specs=[
            pl.BlockSpec((NC, BN, H), lambda i: (0, i, 0)),
            pl.BlockSpec((BN, H), lambda i: (i, 0)),
            pl.BlockSpec((BN, 1), lambda i: (i, 0)),
            pl.BlockSpec((1, H), lambda i: (0, 0)),
            pl.BlockSpec((H, H), lambda i: (0, 0)),
        ],
        out_specs=pl.BlockSpec((BN, H), lambda i: (i, 0)),
        out_shape=jax.ShapeDtypeStruct((N, H), jnp.float32),
    )(acc, g, dinv, b, w)


def _final_body(acc_ref, g_ref, dinv_ref, b_ref, wl_ref, bl_ref, batch_ref,
                out_ref, zacc_ref, cacc_ref):
    i = pl.program_id(0)
    t = acc_ref[0] + acc_ref[1] + g_ref[...]
    h = jnp.maximum(dinv_ref[...] * t + b_ref[...], 0.0)
    z = jnp.dot(h, wl_ref[...], preferred_element_type=jnp.float32,
                precision=lax.Precision.HIGHEST)
    onehot = (
        batch_ref[...] == lax.broadcasted_iota(jnp.int32, (BN, G), 1)
    ).astype(jnp.float32)
    zp = jnp.sum(onehot * z, axis=0, keepdims=True)
    cp = jnp.sum(onehot, axis=0, keepdims=True)

    @pl.when(i == 0)
    def _init():
        zacc_ref[...] = jnp.zeros_like(zacc_ref)
        cacc_ref[...] = jnp.zeros_like(cacc_ref)

    zacc_ref[...] += zp
    cacc_ref[...] += cp

    @pl.when(i == GRID - 1)
    def _emit():
        out_ref[...] = (
            zacc_ref[...] / jnp.maximum(cacc_ref[...], 1.0) + bl_ref[...]
        )


def _final(acc, g, dinv, b, wl, bl, batch2):
    return pl.pallas_call(
        _final_body,
        grid=(GRID,),
        in_specs=[
            pl.BlockSpec((NC, BN, H), lambda i: (0, i, 0)),
            pl.BlockSpec((BN, H), lambda i: (i, 0)),
            pl.BlockSpec((BN, 1), lambda i: (i, 0)),
            pl.BlockSpec((1, H), lambda i: (0, 0)),
            pl.BlockSpec((H, 1), lambda i: (0, 0)),
            pl.BlockSpec((1, 1), lambda i: (0, 0)),
            pl.BlockSpec((BN, 1), lambda i: (i, 0)),
        ],
        out_specs=pl.BlockSpec((1, G), lambda i: (0, 0)),
        out_shape=jax.ShapeDtypeStruct((1, G), jnp.float32),
        scratch_shapes=[
            pltpu.VMEM((1, G), jnp.float32),
            pltpu.VMEM((1, G), jnp.float32),
        ],
    )(acc, g, dinv, b, wl, bl, batch2)


def kernel(x, edge_index, batch, W0, b0, W1, b1, W2, b2, Wl, bl):
    f32 = jnp.float32
    deg_fn, agg_fn = _sc_kernels()
    ones_deg = jnp.ones((CHUNK, DEGW), f32)
    zeros_deg = jnp.zeros((RPT, DEGW), f32)
    zeros_acc = jnp.zeros((RPT, H), f32)

    e_src = edge_index[0]
    e_dst = edge_index[1]
    deg = deg_fn(e_dst, ones_deg, zeros_deg)
    dinv, g1 = _pre(deg, x, W0)
    acc1 = agg_fn(g1, e_src, e_dst, zeros_acc)
    g2 = _layer(acc1, g1, dinv, b0.reshape(1, H), W1)
    acc2 = agg_fn(g2, e_src, e_dst, zeros_acc)
    g3 = _layer(acc2, g2, dinv, b1.reshape(1, H), W2)
    acc3 = agg_fn(g3, e_src, e_dst, zeros_acc)
    out = _final(
        acc3, g3, dinv, b2.reshape(1, H), Wl, bl.reshape(1, 1),
        batch.reshape(N, 1)
    )
    return out.reshape(G)
